# Initial kernel scaffold; baseline (speedup 1.0000x reference)
import jax
import jax.numpy as jnp
from jax.experimental import pallas as pl


def kernel(x, pos, batch, W1, b1, W2, b2):
    raise NotImplementedError("write your pallas kernel here")


# speedup vs baseline: 38.0688x; 38.0688x over previous
import functools

import jax
import jax.numpy as jnp
import numpy as np
from jax import lax
from jax.experimental import pallas as pl
from jax.experimental.pallas import tpu as pltpu
from jax.experimental.pallas import tpu_sc as plsc

R = 0.8
K = 64
NUM_BATCH = 16
R2 = np.float32(R * R)
R2_BITS = int(np.float32(R * R).view(np.int32))

NC, NS, LANES = 2, 16, 16
NW = NC * NS

QGRP = 16
GRP_PER_W = 20


def _a_body(xb, pb, w1x, w1p, b1r, wb, qb):
  qv = jnp.dot(pb[...], w1p[...], preferred_element_type=jnp.float32)
  wb[...] = (
      jnp.dot(xb[...], w1x[...], preferred_element_type=jnp.float32)
      + qv + b1r[...]
  )
  qb[...] = qv


def _precompute_wq(x, posp, W1x, W1p, b1r):
  n, d = x.shape
  bq = 1000
  grid = n // bq
  return pl.pallas_call(
      _a_body,
      grid=(grid,),
      in_specs=[
          pl.BlockSpec((bq, d), lambda i: (i, 0)),
          pl.BlockSpec((bq, 8), lambda i: (i, 0)),
          pl.BlockSpec((d, 128), lambda i: (0, 0)),
          pl.BlockSpec((8, 128), lambda i: (0, 0)),
          pl.BlockSpec((1, 128), lambda i: (0, 0)),
      ],
      out_specs=[
          pl.BlockSpec((bq, 128), lambda i: (i, 0)),
          pl.BlockSpec((bq, 128), lambda i: (i, 0)),
      ],
      out_shape=[
          jax.ShapeDtypeStruct((n, 128), jnp.float32),
          jax.ShapeDtypeStruct((n, 128), jnp.float32),
      ],
  )(x, posp, W1x, W1p, b1r)


def _sel_body(n, px_h, py_h, pz_h, bt_h, idx_h,
              px, py, pz, bt, seglo, seghi, cd, cj, stage):
  cid = lax.axis_index("c")
  sid = lax.axis_index("s")
  wid = sid * NC + cid

  pltpu.sync_copy(px_h, px.at[pl.ds(0, n)])
  pltpu.sync_copy(py_h, py.at[pl.ds(0, n)])
  pltpu.sync_copy(pz_h, pz.at[pl.ds(0, n)])
  pltpu.sync_copy(bt_h, bt.at[pl.ds(0, n)])

  lane = jax.lax.iota(jnp.int32, LANES)

  def _cnt_step(c, carry):
    bv = bt[pl.ds(c * LANES, LANES)]
    out = []
    for b in range(NUM_BATCH):
      m = bv == b
      out.append(carry[b] + jnp.max(plsc.all_reduce_population_count(m)))
    return tuple(out)

  counts = lax.fori_loop(0, n // LANES, _cnt_step,
                         tuple(jnp.int32(0) for _ in range(NUM_BATCH)))
  acc = jnp.int32(0)
  for b in range(NUM_BATCH):
    seglo[b] = acc
    acc = acc + counts[b]
    seghi[b] = acc

  def _per_query(qq, gbase):
    i = gbase + qq
    b = bt[pl.ds(i, LANES)][0]
    lo = seglo[b]
    hi = seghi[b]
    pix = px[pl.ds(i, LANES)][0]
    piy = py[pl.ds(i, LANES)][0]
    piz = pz[pl.ds(i, LANES)][0]
    c0 = lo // LANES
    c1 = (hi + LANES - 1) // LANES

    def _scan_chunk(cc, cnt):
      base = cc * LANES
      jv = base + lane
      dx = px[pl.ds(base, LANES)] - pix
      dy = py[pl.ds(base, LANES)] - piy
      dz = pz[pl.ds(base, LANES)] - piz
      d2 = dx * dx + dy * dy + dz * dz
      m = (d2 <= R2) & (jv >= lo) & (jv < hi)
      plsc.store_compressed(cd.at[pl.ds(cnt, LANES)],
                            plsc.bitcast(d2, jnp.int32), mask=m)
      plsc.store_compressed(cj.at[pl.ds(cnt, LANES)], jv, mask=m)
      return cnt + jnp.max(plsc.all_reduce_population_count(m))

    cnt = lax.fori_loop(c0, c1, _scan_chunk, jnp.int32(0))
    nch = (cnt + LANES - 1) // LANES

    def _find_t(_):
      def _bs_step(_, lh):
        blo, bhi = lh
        mid = (blo + bhi) >> 1

        def _cnt_chunk(k, cv):
          dv = cd[pl.ds(k * LANES, LANES)]
          mm = (dv <= mid) & ((k * LANES + lane) < cnt)
          return cv + plsc.all_reduce_population_count(mm)

        cmid = jnp.max(lax.fori_loop(0, nch, _cnt_chunk,
                                     jnp.zeros((LANES,), jnp.int32)))
        take = cmid >= K
        return (jnp.where(take, blo, mid + 1), jnp.where(take, mid, bhi))

      blo, bhi = lax.fori_loop(0, 31, _bs_step,
                               (jnp.int32(0), jnp.int32(R2_BITS)))
      return bhi

    t = lax.cond(cnt > K, _find_t, lambda _: jnp.int32(R2_BITS),
                 operand=jnp.int32(0))

    rowoff = qq * K
    for c in range(K // LANES):
      stage[pl.ds(rowoff + c * LANES, LANES)] = jnp.full((LANES,), i,
                                                         jnp.int32)

    def _emit_chunk(k, wp):
      dv = cd[pl.ds(k * LANES, LANES)]
      jv = cj[pl.ds(k * LANES, LANES)]
      m2 = (dv <= t) & ((k * LANES + lane) < cnt)
      pc = plsc.cumsum(m2.astype(jnp.int32))
      m3 = m2 & ((wp + pc) <= K)
      plsc.store_compressed(stage.at[pl.ds(rowoff + wp, LANES)], jv, mask=m3)
      return wp + jnp.max(plsc.all_reduce_population_count(m3))

    lax.fori_loop(0, nch, _emit_chunk, jnp.int32(0))
    return gbase

  def _per_group(g, _):
    gbase = wid * (QGRP * GRP_PER_W) + g * QGRP

    @pl.when(gbase < n)
    def _():
      lax.fori_loop(0, QGRP, _per_query, gbase)
      pltpu.sync_copy(stage.at[pl.ds(0, QGRP * K)],
                      idx_h.at[pl.ds(gbase * K, QGRP * K)])

    return 0

  lax.fori_loop(0, GRP_PER_W, _per_group, 0)


def _select_neighbors(posx, posy, posz, batch):
  n = posx.shape[0]
  mesh = plsc.VectorSubcoreMesh(core_axis_name="c", subcore_axis_name="s",
                                num_cores=NC, num_subcores=NS)
  return pl.kernel(
      functools.partial(_sel_body, n),
      out_type=jax.ShapeDtypeStruct((n * K,), jnp.int32),
      mesh=mesh,
      compiler_params=pltpu.CompilerParams(needs_layout_passes=False),
      scratch_types=[
          pltpu.VMEM((n + LANES,), jnp.float32),
          pltpu.VMEM((n + LANES,), jnp.float32),
          pltpu.VMEM((n + LANES,), jnp.float32),
          pltpu.VMEM((n + LANES,), jnp.int32),
          pltpu.SMEM((NUM_BATCH,), jnp.int32),
          pltpu.SMEM((NUM_BATCH,), jnp.int32),
          pltpu.VMEM((n + LANES,), jnp.int32),
          pltpu.VMEM((n + LANES,), jnp.int32),
          pltpu.VMEM((QGRP * K + LANES,), jnp.int32),
      ],
  )(posx, posy, posz, batch)


GCH = 128


def _gather_body(e_per_w, n_full, tail, w_h, idxf_h, wg_h,
                 idxv, rows0, rows1, sem0, sem1):
  cid = lax.axis_index("c")
  sid = lax.axis_index("s")
  wid = sid * NC + cid
  base = wid * e_per_w

  pltpu.sync_copy(idxf_h.at[pl.ds(base, e_per_w)], idxv)

  def _fire(g, rows, sem):
    pltpu.async_copy(w_h.at[idxv.at[pl.ds(g * GCH, GCH)]], rows, sem)

  def _drain_store(g, rows, sem):
    pltpu.make_async_copy(w_h.at[idxv.at[pl.ds(g * GCH, GCH)]], rows,
                          sem).wait()
    pltpu.sync_copy(rows, wg_h.at[pl.ds(base + g * GCH, GCH)])

  _fire(0, rows0, sem0)
  if n_full > 1:
    _fire(1, rows1, sem1)

  def _step(h, _):
    g0 = 2 * h

    @pl.when(g0 + 2 < n_full)
    def _():
      _drain_store(g0, rows0, sem0)
      _fire(g0 + 2, rows0, sem0)

    @pl.when(g0 + 2 >= n_full)
    def _():
      _drain_store(g0, rows0, sem0)

    @pl.when(g0 + 3 < n_full)
    def _():
      _drain_store(g0 + 1, rows1, sem1)
      _fire(g0 + 3, rows1, sem1)

    @pl.when((g0 + 1 < n_full) & (g0 + 3 >= n_full))
    def _():
      _drain_store(g0 + 1, rows1, sem1)

    return 0

  lax.fori_loop(0, (n_full + 1) // 2, _step, 0)

  if tail:
    pltpu.async_copy(
        w_h.at[idxv.at[pl.ds(n_full * GCH, tail)]],
        rows0.at[pl.ds(0, tail)], sem0)
    pltpu.make_async_copy(
        w_h.at[idxv.at[pl.ds(n_full * GCH, tail)]],
        rows0.at[pl.ds(0, tail)], sem0).wait()
    pltpu.sync_copy(rows0.at[pl.ds(0, tail)],
                    wg_h.at[pl.ds(base + n_full * GCH, tail)])


def _gather_rows(w, idxf):
  e = idxf.shape[0]
  e_per_w = e // NW
  n_full = e_per_w // GCH
  tail = e_per_w - n_full * GCH
  mesh = plsc.VectorSubcoreMesh(core_axis_name="c", subcore_axis_name="s",
                                num_cores=NC, num_subcores=NS)
  return pl.kernel(
      functools.partial(_gather_body, e_per_w, n_full, tail),
      out_type=jax.ShapeDtypeStruct((e, 128), jnp.float32),
      mesh=mesh,
      scratch_types=[
          pltpu.VMEM((e_per_w,), jnp.int32),
          pltpu.VMEM((GCH, 128), jnp.float32),
          pltpu.VMEM((GCH, 128), jnp.float32),
          pltpu.SemaphoreType.DMA,
          pltpu.SemaphoreType.DMA,
      ],
  )(w, idxf)


def _c_body(wg_b, q_b, w2, b2r, out_b):
  qc = q_b.shape[0]
  h1 = jnp.maximum(wg_b[...] - q_b[...][:, None, :], 0.0)
  h2 = jnp.dot(h1.reshape(qc * K, 128), w2[...],
               preferred_element_type=jnp.float32) + b2r[...]
  h2 = jnp.maximum(h2, 0.0)
  out_b[...] = jnp.max(h2.reshape(qc, K, 128), axis=1)


def _conv_max(wg, q, W2, b2r):
  n = q.shape[0]
  qc = 200
  grid = n // qc
  return pl.pallas_call(
      _c_body,
      grid=(grid,),
      in_specs=[
          pl.BlockSpec((qc, K, 128), lambda i: (i, 0, 0)),
          pl.BlockSpec((qc, 128), lambda i: (i, 0)),
          pl.BlockSpec((128, 128), lambda i: (0, 0)),
          pl.BlockSpec((1, 128), lambda i: (0, 0)),
      ],
      out_specs=pl.BlockSpec((qc, 128), lambda i: (i, 0)),
      out_shape=jax.ShapeDtypeStruct((n, 128), jnp.float32),
  )(wg, q, W2, b2r)


def kernel(x, pos, batch, W1, b1, W2, b2):
  n, d = x.shape
  posp = jnp.pad(pos, ((0, 0), (0, 5)))
  W1x = W1[:d]
  W1p = jnp.pad(W1[d:], ((0, 5), (0, 0)))
  b1r = b1.reshape(1, 128)
  b2r = b2.reshape(1, 128)

  w, q = _precompute_wq(x, posp, W1x, W1p, b1r)
  idx = _select_neighbors(pos[:, 0], pos[:, 1], pos[:, 2],
                          batch.astype(jnp.int32))
  wg = _gather_rows(w, idx)
  out = _conv_max(wg.reshape(n, K, 128), q, W2, b2r)
  return (out, pos, batch)
